# initial kernel scaffold (unmeasured)
import jax
import jax.numpy as jnp
from jax import lax
from jax.experimental import pallas as pl
from jax.experimental.pallas import tpu as pltpu

N_DEV = 4
M = 4096
KS = 1024
N = 8192
MC = M // N_DEV
NB = 1024
N_BLOCKS = N // NB


def kernel(x, w_mat, scale_x, scale_w):
    sxw = (scale_x.reshape(()) * scale_w.reshape(())).reshape(1, 1)
    x8 = x.astype(jnp.float8_e5m2)
    w8 = w_mat.astype(jnp.float8_e5m2)

    def body(sxw_ref, x_ref, w_ref, out_ref,
             send_buf, recv_buf, send_sem, recv_sem, credit_sem):
        my = lax.axis_index("i")
        left = lax.rem(my + (N_DEV - 1), N_DEV)
        right = lax.rem(my + 1, N_DEV)

        barrier_sem = pltpu.get_barrier_semaphore()
        for nbr in [left, right]:
            pl.semaphore_signal(
                barrier_sem, inc=1,
                device_id=(nbr,), device_id_type=pl.DeviceIdType.MESH,
            )
        pl.semaphore_wait(barrier_sem, 2)

        sxw = sxw_ref[0, 0]

        def partial_chunk(c, nlo, nhi):
            xc = x_ref[pl.ds(c * MC, MC), :]
            wb = w_ref[:, nlo:nhi]
            return lax.dot_general(
                xc, wb,
                dimension_numbers=(((1,), (0,)), ((), ())),
                preferred_element_type=jnp.float32,
            )

        for b in range(N_BLOCKS):
            nlo, nhi = b * NB, (b + 1) * NB
            c0 = lax.rem(my + (N_DEV - 1), N_DEV)
            send_buf[...] = partial_chunk(c0, nlo, nhi).astype(jnp.bfloat16)

            for s in range(N_DEV - 1):
                if not (b == 0 and s == 0):
                    pl.semaphore_wait(credit_sem, 1)
                rdma = pltpu.make_async_remote_copy(
                    src_ref=send_buf,
                    dst_ref=recv_buf,
                    send_sem=send_sem,
                    recv_sem=recv_sem,
                    device_id=(right,),
                    device_id_type=pl.DeviceIdType.MESH,
                )
                rdma.start()
                rdma.wait()

                c = lax.rem(my + (2 * N_DEV - 2 - s), N_DEV)
                acc = partial_chunk(c, nlo, nhi) + recv_buf[...].astype(jnp.float32)
                if s < N_DEV - 2:
                    send_buf[...] = acc.astype(jnp.bfloat16)
                else:
                    out_ref[:, nlo:nhi] = jnp.maximum(acc * sxw, 0.0)
                pl.semaphore_signal(
                    credit_sem, inc=1,
                    device_id=(left,), device_id_type=pl.DeviceIdType.MESH,
                )

        pl.semaphore_wait(credit_sem, 1)

    return pl.pallas_call(
        body,
        out_shape=jax.ShapeDtypeStruct((MC, N), jnp.float32),
        in_specs=[
            pl.BlockSpec(memory_space=pltpu.SMEM),
            pl.BlockSpec(memory_space=pltpu.VMEM),
            pl.BlockSpec(memory_space=pltpu.VMEM),
        ],
        out_specs=pl.BlockSpec(memory_space=pltpu.VMEM),
        scratch_shapes=[
            pltpu.VMEM((MC, NB), jnp.bfloat16),
            pltpu.VMEM((MC, NB), jnp.bfloat16),
            pltpu.SemaphoreType.DMA,
            pltpu.SemaphoreType.DMA,
            pltpu.SemaphoreType.REGULAR,
        ],
        compiler_params=pltpu.CompilerParams(collective_id=0),
    )(sxw, x8, w8)


# baseline (device time: 701008 ns/iter reference)
import jax
import jax.numpy as jnp
from jax import lax
from jax.experimental import pallas as pl
from jax.experimental.pallas import tpu as pltpu

N_DEV = 4
M = 4096
KS = 1024
N = 8192
MC = M // N_DEV
NB = 1024
N_BLOCKS = N // NB


def kernel(x, w_mat, scale_x, scale_w):
    sxw = (scale_x.reshape(()) * scale_w.reshape(())).reshape(1, 1)
    x8 = x.astype(jnp.float8_e5m2)
    w8 = w_mat.astype(jnp.float8_e5m2)

    def body(sxw_ref, x_ref, w_ref, out_ref,
             send_buf, recv_buf, send_sem, recv_sem, credit_sem):
        my = lax.axis_index("i")
        left = lax.rem(my + (N_DEV - 1), N_DEV)
        right = lax.rem(my + 1, N_DEV)

        barrier_sem = pltpu.get_barrier_semaphore()
        for nbr in [left, right]:
            pl.semaphore_signal(
                barrier_sem, inc=1,
                device_id=(nbr,), device_id_type=pl.DeviceIdType.MESH,
            )
        pl.semaphore_wait(barrier_sem, 2)

        sxw = sxw_ref[0, 0]

        def partial_chunk(c, nlo, nhi):
            xc = x_ref[pl.ds(c * MC, MC), :]
            wb = w_ref[:, nlo:nhi]
            return lax.dot_general(
                xc, wb,
                dimension_numbers=(((1,), (0,)), ((), ())),
                preferred_element_type=jnp.float32,
            )

        for b in range(N_BLOCKS):
            nlo, nhi = b * NB, (b + 1) * NB
            c0 = lax.rem(my + (N_DEV - 1), N_DEV)
            send_buf[...] = partial_chunk(c0, nlo, nhi).astype(jnp.bfloat16)

            for s in range(N_DEV - 1):
                if not (b == 0 and s == 0):
                    pl.semaphore_wait(credit_sem, 1)
                rdma = pltpu.make_async_remote_copy(
                    src_ref=send_buf,
                    dst_ref=recv_buf,
                    send_sem=send_sem,
                    recv_sem=recv_sem,
                    device_id=(right,),
                    device_id_type=pl.DeviceIdType.MESH,
                )
                rdma.start()
                rdma.wait()

                c = lax.rem(my + (2 * N_DEV - 2 - s), N_DEV)
                acc = partial_chunk(c, nlo, nhi) + recv_buf[...].astype(jnp.float32)
                if s < N_DEV - 2:
                    send_buf[...] = acc.astype(jnp.bfloat16)
                else:
                    out_ref[:, nlo:nhi] = jnp.maximum(acc * sxw, 0.0)
                pl.semaphore_signal(
                    credit_sem, inc=1,
                    device_id=(left,), device_id_type=pl.DeviceIdType.MESH,
                )

        pl.semaphore_wait(credit_sem, 1)

    return pl.pallas_call(
        body,
        out_shape=jax.ShapeDtypeStruct((MC, N), jnp.float32),
        in_specs=[
            pl.BlockSpec(memory_space=pltpu.SMEM),
            pl.BlockSpec(memory_space=pltpu.VMEM),
            pl.BlockSpec(memory_space=pltpu.VMEM),
        ],
        out_specs=pl.BlockSpec(memory_space=pltpu.VMEM),
        scratch_shapes=[
            pltpu.VMEM((MC, NB), jnp.bfloat16),
            pltpu.VMEM((MC, NB), jnp.bfloat16),
            pltpu.SemaphoreType.DMA,
            pltpu.SemaphoreType.DMA,
            pltpu.SemaphoreType.REGULAR,
        ],
        compiler_params=pltpu.CompilerParams(
            collective_id=0,
            vmem_limit_bytes=48 * 1024 * 1024,
        ),
    )(sxw, x8, w8)


# device time: 330648 ns/iter; 2.1201x vs baseline; 2.1201x over previous
import jax
import jax.numpy as jnp
from jax import lax
from jax.experimental import pallas as pl
from jax.experimental.pallas import tpu as pltpu

N_DEV = 4
M = 4096
KS = 1024
N = 8192
MC = M // N_DEV
NB = 1024
NH = NB // 2
N_BLOCKS = N // NB


def kernel(x, w_mat, scale_x, scale_w):
    sxw = (scale_x.reshape(()) * scale_w.reshape(())).reshape(1, 1)
    x8 = x.astype(jnp.float8_e5m2)
    w8 = w_mat.astype(jnp.float8_e5m2)

    def body(sxw_ref, x_ref, w_ref, out_ref,
             send_bufs, recv_bufs, staging,
             send_sems, recv_sems, copy_sems, credit_r, credit_l):
        my = lax.axis_index("i")
        left = lax.rem(my + (N_DEV - 1), N_DEV)
        right = lax.rem(my + 1, N_DEV)

        barrier_sem = pltpu.get_barrier_semaphore()
        for nbr in [left, right]:
            pl.semaphore_signal(
                barrier_sem, inc=1,
                device_id=(nbr,), device_id_type=pl.DeviceIdType.MESH,
            )
        pl.semaphore_wait(barrier_sem, 2)

        sxw = sxw_ref[0, 0]

        dsts = [right, left]
        ups = [left, right]
        credits = [credit_r, credit_l]

        def partial_chunk(c, col):
            xc = x_ref[pl.ds(c * MC, MC), :]
            wb = w_ref[:, col:col + NH]
            return lax.dot_general(
                xc, wb,
                dimension_numbers=(((1,), (0,)), ((), ())),
                preferred_element_type=jnp.float32,
            )

        def make_rdma(di, b):
            return pltpu.make_async_remote_copy(
                src_ref=send_bufs.at[di, b],
                dst_ref=recv_bufs.at[di, b],
                send_sem=send_sems.at[di, b],
                recv_sem=recv_sems.at[di, b],
                device_id=(dsts[di],),
                device_id_type=pl.DeviceIdType.MESH,
            )

        rd = [[None] * N_BLOCKS for _ in range(2)]

        c_seed = [lax.rem(my + (N_DEV - 1), N_DEV), lax.rem(my + 1, N_DEV)]
        for b in range(N_BLOCKS):
            for di in range(2):
                col = b * NB + di * NH
                send_bufs[di, b] = partial_chunk(c_seed[di], col).astype(jnp.bfloat16)
                r = make_rdma(di, b)
                r.start()
                rd[di][b] = r

        for s in (1, 2):
            c_cons = [lax.rem(my + (2 * N_DEV - 1 - s), N_DEV),
                      lax.rem(my + s + 1, N_DEV)]
            for b in range(N_BLOCKS):
                for di in range(2):
                    col = b * NB + di * NH
                    part = partial_chunk(c_cons[di], col)
                    prev = rd[di][b]
                    prev.wait_recv()
                    acc = part + recv_bufs[di, b].astype(jnp.float32)
                    prev.wait_send()
                    send_bufs[di, b] = acc.astype(jnp.bfloat16)
                    pl.semaphore_signal(
                        credits[di], inc=1,
                        device_id=(ups[di],),
                        device_id_type=pl.DeviceIdType.MESH,
                    )
                    pl.semaphore_wait(credits[di], 1)
                    r = make_rdma(di, b)
                    r.start()
                    rd[di][b] = r

        copies = [None, None]
        for b in range(N_BLOCKS):
            slot = b % 2
            results = []
            for di in range(2):
                col = b * NB + di * NH
                part = partial_chunk(my, col)
                prev = rd[di][b]
                prev.wait_recv()
                acc = part + recv_bufs[di, b].astype(jnp.float32)
                results.append(jnp.maximum(acc * sxw, 0.0))
                prev.wait_send()
            if copies[slot] is not None:
                copies[slot].wait()
            staging[slot, :, :NH] = results[0]
            staging[slot, :, NH:] = results[1]
            cp = pltpu.make_async_copy(
                staging.at[slot],
                out_ref.at[:, pl.ds(b * NB, NB)],
                copy_sems.at[slot],
            )
            cp.start()
            copies[slot] = cp
        for cp in copies:
            cp.wait()

    return pl.pallas_call(
        body,
        out_shape=jax.ShapeDtypeStruct((MC, N), jnp.float32),
        in_specs=[
            pl.BlockSpec(memory_space=pltpu.SMEM),
            pl.BlockSpec(memory_space=pltpu.VMEM),
            pl.BlockSpec(memory_space=pltpu.VMEM),
        ],
        out_specs=pl.BlockSpec(memory_space=pl.ANY),
        scratch_shapes=[
            pltpu.VMEM((2, N_BLOCKS, MC, NH), jnp.bfloat16),
            pltpu.VMEM((2, N_BLOCKS, MC, NH), jnp.bfloat16),
            pltpu.VMEM((2, MC, NB), jnp.float32),
            pltpu.SemaphoreType.DMA((2, N_BLOCKS)),
            pltpu.SemaphoreType.DMA((2, N_BLOCKS)),
            pltpu.SemaphoreType.DMA((2,)),
            pltpu.SemaphoreType.REGULAR,
            pltpu.SemaphoreType.REGULAR,
        ],
        compiler_params=pltpu.CompilerParams(
            collective_id=0,
            vmem_limit_bytes=56 * 1024 * 1024,
        ),
    )(sxw, x8, w8)


# device time: 314687 ns/iter; 2.2276x vs baseline; 1.0507x over previous
import jax
import jax.numpy as jnp
from jax import lax
from jax.experimental import pallas as pl
from jax.experimental.pallas import tpu as pltpu

N_DEV = 4
M = 4096
KS = 1024
N = 8192
MC = M // N_DEV
NB = 1024
NH = NB // 2
N_BLOCKS = N // NB
N_SLOTS = 4


def kernel(x, w_mat, scale_x, scale_w):
    sxw = (scale_x.reshape(()) * scale_w.reshape(())).reshape(1, 1)
    x8 = x.astype(jnp.float8_e5m2)

    def body(sxw_ref, x_ref, w_hbm, out_ref,
             w8, wstage, send_bufs, recv_bufs, staging,
             wdma_sems, send_sems, recv_sems, copy_sem, credit_r, credit_l):
        my = lax.axis_index("i")
        left = lax.rem(my + (N_DEV - 1), N_DEV)
        right = lax.rem(my + 1, N_DEV)

        barrier_sem = pltpu.get_barrier_semaphore()
        for nbr in [left, right]:
            pl.semaphore_signal(
                barrier_sem, inc=1,
                device_id=(nbr,), device_id_type=pl.DeviceIdType.MESH,
            )
        pl.semaphore_wait(barrier_sem, 2)

        sxw = sxw_ref[0, 0]

        dsts = [right, left]
        ups = [left, right]
        credits = [credit_r, credit_l]

        def partial_chunk(c, col):
            xc = x_ref[pl.ds(c * MC, MC), :]
            wb = w8[:, col:col + NH]
            return lax.dot_general(
                xc, wb,
                dimension_numbers=(((1,), (0,)), ((), ())),
                preferred_element_type=jnp.float32,
            )

        def make_rdma(di, b):
            return pltpu.make_async_remote_copy(
                src_ref=send_bufs.at[di, b % N_SLOTS],
                dst_ref=recv_bufs.at[di, b],
                send_sem=send_sems.at[di, b % N_SLOTS],
                recv_sem=recv_sems.at[di, b],
                device_id=(dsts[di],),
                device_id_type=pl.DeviceIdType.MESH,
            )

        def w_fetch(b, do_wait=True):
            cp = pltpu.make_async_copy(
                w_hbm.at[:, pl.ds(b * NB, NB)],
                wstage.at[b % 2],
                wdma_sems.at[b % 2],
            )
            return cp

        rd = [[None] * N_BLOCKS for _ in range(2)]
        rd_slot = [[None] * N_SLOTS for _ in range(2)]

        wdma = [None] * N_BLOCKS
        for b in (0, 1):
            wdma[b] = w_fetch(b)
            wdma[b].start()

        c_seed = [lax.rem(my + (N_DEV - 1), N_DEV), lax.rem(my + 1, N_DEV)]
        for b in range(N_BLOCKS):
            wdma[b].wait()
            w8[:, b * NB:(b + 1) * NB] = wstage[b % 2].astype(jnp.float8_e5m2)
            if b + 2 < N_BLOCKS:
                wdma[b + 2] = w_fetch(b + 2)
                wdma[b + 2].start()
            for di in range(2):
                col = b * NB + di * NH
                if rd_slot[di][b % N_SLOTS] is not None:
                    rd_slot[di][b % N_SLOTS].wait_send()
                send_bufs[di, b % N_SLOTS] = (
                    partial_chunk(c_seed[di], col).astype(jnp.bfloat16))
                r = make_rdma(di, b)
                r.start()
                rd[di][b] = r
                rd_slot[di][b % N_SLOTS] = r

        for s in (1, 2):
            c_cons = [lax.rem(my + (2 * N_DEV - 1 - s), N_DEV),
                      lax.rem(my + s + 1, N_DEV)]
            for b in range(N_BLOCKS):
                parts = [partial_chunk(c_cons[di], b * NB + di * NH)
                         for di in range(2)]
                for di in range(2):
                    prev = rd[di][b]
                    prev.wait_recv()
                    acc = parts[di] + recv_bufs[di, b].astype(jnp.float32)
                    if rd_slot[di][b % N_SLOTS] is not None:
                        rd_slot[di][b % N_SLOTS].wait_send()
                    send_bufs[di, b % N_SLOTS] = acc.astype(jnp.bfloat16)
                    pl.semaphore_signal(
                        credits[di], inc=1,
                        device_id=(ups[di],),
                        device_id_type=pl.DeviceIdType.MESH,
                    )
                    pl.semaphore_wait(credits[di], 1)
                    r = make_rdma(di, b)
                    r.start()
                    rd[di][b] = r
                    rd_slot[di][b % N_SLOTS] = r

        last_copy = None
        for b in range(N_BLOCKS):
            parts = [partial_chunk(my, b * NB + di * NH) for di in range(2)]
            results = []
            for di in range(2):
                prev = rd[di][b]
                prev.wait_recv()
                acc = parts[di] + recv_bufs[di, b].astype(jnp.float32)
                results.append(jnp.maximum(acc * sxw, 0.0))
            if last_copy is not None:
                last_copy.wait()
            staging[:, :NH] = results[0]
            staging[:, NH:] = results[1]
            cp = pltpu.make_async_copy(
                staging,
                out_ref.at[:, pl.ds(b * NB, NB)],
                copy_sem,
            )
            cp.start()
            last_copy = cp
        last_copy.wait()

        for di in range(2):
            for sl in range(N_SLOTS):
                rd_slot[di][sl].wait_send()

    return pl.pallas_call(
        body,
        out_shape=jax.ShapeDtypeStruct((MC, N), jnp.float32),
        in_specs=[
            pl.BlockSpec(memory_space=pltpu.SMEM),
            pl.BlockSpec(memory_space=pltpu.VMEM),
            pl.BlockSpec(memory_space=pl.ANY),
        ],
        out_specs=pl.BlockSpec(memory_space=pl.ANY),
        scratch_shapes=[
            pltpu.VMEM((KS, N), jnp.float8_e5m2),
            pltpu.VMEM((2, KS, NB), jnp.float32),
            pltpu.VMEM((2, N_SLOTS, MC, NH), jnp.bfloat16),
            pltpu.VMEM((2, N_BLOCKS, MC, NH), jnp.bfloat16),
            pltpu.VMEM((MC, NB), jnp.float32),
            pltpu.SemaphoreType.DMA((2,)),
            pltpu.SemaphoreType.DMA((2, N_SLOTS)),
            pltpu.SemaphoreType.DMA((2, N_BLOCKS)),
            pltpu.SemaphoreType.DMA,
            pltpu.SemaphoreType.REGULAR,
            pltpu.SemaphoreType.REGULAR,
        ],
        compiler_params=pltpu.CompilerParams(
            collective_id=0,
            vmem_limit_bytes=60 * 1024 * 1024,
        ),
    )(sxw, x8, w_mat)


# device time: 307443 ns/iter; 2.2801x vs baseline; 1.0236x over previous
import jax
import jax.numpy as jnp
from jax import lax
from jax.experimental import pallas as pl
from jax.experimental.pallas import tpu as pltpu

N_DEV = 4
M = 4096
KS = 1024
N = 8192
MC = M // N_DEV
NB = 1024
NH = NB // 2
N_BLOCKS = N // NB
N_SLOTS = 4


def kernel(x, w_mat, scale_x, scale_w):
    sxw = (scale_x.reshape(()) * scale_w.reshape(())).reshape(1, 1)

    def body(sxw_ref, x_hbm, w_hbm, out_ref,
             x8, xstage, w8, wstage, send_bufs, recv_bufs, staging,
             xdma_sems, wdma_sems, send_sems, recv_sems, copy_sem,
             credit_r, credit_l):
        my = lax.axis_index("i")
        left = lax.rem(my + (N_DEV - 1), N_DEV)
        right = lax.rem(my + 1, N_DEV)

        barrier_sem = pltpu.get_barrier_semaphore()
        for nbr in [left, right]:
            pl.semaphore_signal(
                barrier_sem, inc=1,
                device_id=(nbr,), device_id_type=pl.DeviceIdType.MESH,
            )
        pl.semaphore_wait(barrier_sem, 2)

        sxw = sxw_ref[0, 0]

        dsts = [right, left]
        ups = [left, right]
        credits = [credit_r, credit_l]

        def partial_chunk(c, col):
            xc = x8[pl.ds(c * MC, MC), :]
            wb = w8[:, col:col + NH]
            return lax.dot_general(
                xc, wb,
                dimension_numbers=(((1,), (0,)), ((), ())),
                preferred_element_type=jnp.float32,
            )

        def make_rdma(di, b):
            return pltpu.make_async_remote_copy(
                src_ref=send_bufs.at[di, b % N_SLOTS],
                dst_ref=recv_bufs.at[di, b],
                send_sem=send_sems.at[di, b % N_SLOTS],
                recv_sem=recv_sems.at[di, b],
                device_id=(dsts[di],),
                device_id_type=pl.DeviceIdType.MESH,
            )

        def w_fetch(b, do_wait=True):
            cp = pltpu.make_async_copy(
                w_hbm.at[:, pl.ds(b * NB, NB)],
                wstage.at[b % 2],
                wdma_sems.at[b % 2],
            )
            return cp

        def x_fetch(c, slot):
            return pltpu.make_async_copy(
                x_hbm.at[pl.ds(c * MC, MC), :],
                xstage.at[slot],
                xdma_sems.at[slot],
            )

        def x_cast(c, slot):
            x8[pl.ds(c * MC, MC), :] = xstage[slot].astype(jnp.float8_e5m2)

        rd = [[None] * N_BLOCKS for _ in range(2)]
        rd_slot = [[None] * N_SLOTS for _ in range(2)]

        c_seed = [lax.rem(my + (N_DEV - 1), N_DEV), lax.rem(my + 1, N_DEV)]
        c_mid = lax.rem(my + 2, N_DEV)
        xdma = [x_fetch(c_seed[0], 0), x_fetch(c_seed[1], 1)]
        for cp in xdma:
            cp.start()

        wdma = [None] * N_BLOCKS
        for b in (0, 1):
            wdma[b] = w_fetch(b)
            wdma[b].start()

        for slot in (0, 1):
            xdma[slot].wait()
            x_cast(c_seed[slot], slot)
        xdma = [x_fetch(c_mid, 0), x_fetch(my, 1)]
        for cp in xdma:
            cp.start()

        for b in range(N_BLOCKS):
            wdma[b].wait()
            w8[:, b * NB:(b + 1) * NB] = wstage[b % 2].astype(jnp.float8_e5m2)
            if b + 2 < N_BLOCKS:
                wdma[b + 2] = w_fetch(b + 2)
                wdma[b + 2].start()
            for di in range(2):
                col = b * NB + di * NH
                if rd_slot[di][b % N_SLOTS] is not None:
                    rd_slot[di][b % N_SLOTS].wait_send()
                send_bufs[di, b % N_SLOTS] = (
                    partial_chunk(c_seed[di], col).astype(jnp.bfloat16))
                r = make_rdma(di, b)
                r.start()
                rd[di][b] = r
                rd_slot[di][b % N_SLOTS] = r

        for s in (1, 2):
            if s == 1:
                xdma[0].wait()
                x_cast(c_mid, 0)
            c_cons = [lax.rem(my + (2 * N_DEV - 1 - s), N_DEV),
                      lax.rem(my + s + 1, N_DEV)]
            for b in range(N_BLOCKS):
                parts = [partial_chunk(c_cons[di], b * NB + di * NH)
                         for di in range(2)]
                for di in range(2):
                    prev = rd[di][b]
                    prev.wait_recv()
                    acc = parts[di] + recv_bufs[di, b].astype(jnp.float32)
                    if rd_slot[di][b % N_SLOTS] is not None:
                        rd_slot[di][b % N_SLOTS].wait_send()
                    send_bufs[di, b % N_SLOTS] = acc.astype(jnp.bfloat16)
                    pl.semaphore_signal(
                        credits[di], inc=1,
                        device_id=(ups[di],),
                        device_id_type=pl.DeviceIdType.MESH,
                    )
                    pl.semaphore_wait(credits[di], 1)
                    r = make_rdma(di, b)
                    r.start()
                    rd[di][b] = r
                    rd_slot[di][b % N_SLOTS] = r

        xdma[1].wait()
        x_cast(my, 1)
        last_copy = None
        for b in range(N_BLOCKS):
            parts = [partial_chunk(my, b * NB + di * NH) for di in range(2)]
            results = []
            for di in range(2):
                prev = rd[di][b]
                prev.wait_recv()
                acc = parts[di] + recv_bufs[di, b].astype(jnp.float32)
                results.append(jnp.maximum(acc * sxw, 0.0))
            if last_copy is not None:
                last_copy.wait()
            staging[:, :NH] = results[0]
            staging[:, NH:] = results[1]
            cp = pltpu.make_async_copy(
                staging,
                out_ref.at[:, pl.ds(b * NB, NB)],
                copy_sem,
            )
            cp.start()
            last_copy = cp
        last_copy.wait()

        for di in range(2):
            for sl in range(N_SLOTS):
                rd_slot[di][sl].wait_send()

    return pl.pallas_call(
        body,
        out_shape=jax.ShapeDtypeStruct((MC, N), jnp.float32),
        in_specs=[
            pl.BlockSpec(memory_space=pltpu.SMEM),
            pl.BlockSpec(memory_space=pl.ANY),
            pl.BlockSpec(memory_space=pl.ANY),
        ],
        out_specs=pl.BlockSpec(memory_space=pl.ANY),
        scratch_shapes=[
            pltpu.VMEM((M, KS), jnp.float8_e5m2),
            pltpu.VMEM((2, MC, KS), jnp.float32),
            pltpu.VMEM((KS, N), jnp.float8_e5m2),
            pltpu.VMEM((2, KS, NB), jnp.float32),
            pltpu.VMEM((2, N_SLOTS, MC, NH), jnp.bfloat16),
            pltpu.VMEM((2, N_BLOCKS, MC, NH), jnp.bfloat16),
            pltpu.VMEM((MC, NB), jnp.float32),
            pltpu.SemaphoreType.DMA((2,)),
            pltpu.SemaphoreType.DMA((2,)),
            pltpu.SemaphoreType.DMA((2, N_SLOTS)),
            pltpu.SemaphoreType.DMA((2, N_BLOCKS)),
            pltpu.SemaphoreType.DMA,
            pltpu.SemaphoreType.REGULAR,
            pltpu.SemaphoreType.REGULAR,
        ],
        compiler_params=pltpu.CompilerParams(
            collective_id=0,
            vmem_limit_bytes=62 * 1024 * 1024,
        ),
    )(sxw, x, w_mat)


# device time: 307403 ns/iter; 2.2804x vs baseline; 1.0001x over previous
import jax
import jax.numpy as jnp
from jax import lax
from jax.experimental import pallas as pl
from jax.experimental.pallas import tpu as pltpu

N_DEV = 4
M = 4096
KS = 1024
N = 8192
MC = M // N_DEV
NB = 1024
NH = NB // 2
N_BLOCKS = N // NB
N_SLOTS = 4


def kernel(x, w_mat, scale_x, scale_w):
    sxw = (scale_x.reshape(()) * scale_w.reshape(())).reshape(1, 1)

    def body(sxw_ref, x_hbm, w_hbm, out_ref,
             x8, xstage, w8, wstage, send_bufs, recv_bufs, staging,
             xdma_sems, wdma_sems, send_sems, recv_sems, copy_sem,
             credit_r, credit_l):
        my = lax.axis_index("i")
        left = lax.rem(my + (N_DEV - 1), N_DEV)
        right = lax.rem(my + 1, N_DEV)

        sxw = sxw_ref[0, 0]

        dsts = [right, left]
        ups = [left, right]
        credits = [credit_r, credit_l]

        def partial_chunk(c, col):
            xc = x8[pl.ds(c * MC, MC), :]
            wb = w8[:, col:col + NH]
            return lax.dot_general(
                xc, wb,
                dimension_numbers=(((1,), (0,)), ((), ())),
                preferred_element_type=jnp.float32,
            )

        def make_rdma(di, b):
            return pltpu.make_async_remote_copy(
                src_ref=send_bufs.at[di, b % N_SLOTS],
                dst_ref=recv_bufs.at[di, b],
                send_sem=send_sems.at[di, b % N_SLOTS],
                recv_sem=recv_sems.at[di, b],
                device_id=(dsts[di],),
                device_id_type=pl.DeviceIdType.MESH,
            )

        def w_fetch(b, do_wait=True):
            cp = pltpu.make_async_copy(
                w_hbm.at[:, pl.ds(b * NB, NB)],
                wstage.at[b % 2],
                wdma_sems.at[b % 2],
            )
            return cp

        def x_fetch(c, slot):
            return pltpu.make_async_copy(
                x_hbm.at[pl.ds(c * MC, MC), :],
                xstage.at[slot],
                xdma_sems.at[slot],
            )

        def x_cast(c, slot):
            x8[pl.ds(c * MC, MC), :] = xstage[slot].astype(jnp.float8_e5m2)

        rd = [[None] * N_BLOCKS for _ in range(2)]
        rd_slot = [[None] * N_SLOTS for _ in range(2)]

        c_seed = [lax.rem(my + (N_DEV - 1), N_DEV), lax.rem(my + 1, N_DEV)]
        c_mid = lax.rem(my + 2, N_DEV)
        xdma = [x_fetch(c_seed[0], 0), x_fetch(c_seed[1], 1)]
        for cp in xdma:
            cp.start()

        wdma = [None] * N_BLOCKS
        for b in (0, 1):
            wdma[b] = w_fetch(b)
            wdma[b].start()

        barrier_sem = pltpu.get_barrier_semaphore()
        for nbr in [left, right]:
            pl.semaphore_signal(
                barrier_sem, inc=1,
                device_id=(nbr,), device_id_type=pl.DeviceIdType.MESH,
            )
        pl.semaphore_wait(barrier_sem, 2)

        for slot in (0, 1):
            xdma[slot].wait()
            x_cast(c_seed[slot], slot)
        xdma = [x_fetch(c_mid, 0), x_fetch(my, 1)]
        for cp in xdma:
            cp.start()

        for b in range(N_BLOCKS):
            wdma[b].wait()
            w8[:, b * NB:(b + 1) * NB] = wstage[b % 2].astype(jnp.float8_e5m2)
            if b + 2 < N_BLOCKS:
                wdma[b + 2] = w_fetch(b + 2)
                wdma[b + 2].start()
            for di in range(2):
                col = b * NB + di * NH
                if rd_slot[di][b % N_SLOTS] is not None:
                    rd_slot[di][b % N_SLOTS].wait_send()
                send_bufs[di, b % N_SLOTS] = (
                    partial_chunk(c_seed[di], col).astype(jnp.bfloat16))
                r = make_rdma(di, b)
                r.start()
                rd[di][b] = r
                rd_slot[di][b % N_SLOTS] = r

        for s in (1, 2):
            if s == 1:
                xdma[0].wait()
                x_cast(c_mid, 0)
            c_cons = [lax.rem(my + (2 * N_DEV - 1 - s), N_DEV),
                      lax.rem(my + s + 1, N_DEV)]
            for b in range(N_BLOCKS):
                parts = [partial_chunk(c_cons[di], b * NB + di * NH)
                         for di in range(2)]
                for di in range(2):
                    prev = rd[di][b]
                    prev.wait_recv()
                    acc = parts[di] + recv_bufs[di, b].astype(jnp.float32)
                    if rd_slot[di][b % N_SLOTS] is not None:
                        rd_slot[di][b % N_SLOTS].wait_send()
                    send_bufs[di, b % N_SLOTS] = acc.astype(jnp.bfloat16)
                    pl.semaphore_signal(
                        credits[di], inc=1,
                        device_id=(ups[di],),
                        device_id_type=pl.DeviceIdType.MESH,
                    )
                    pl.semaphore_wait(credits[di], 1)
                    r = make_rdma(di, b)
                    r.start()
                    rd[di][b] = r
                    rd_slot[di][b % N_SLOTS] = r

        xdma[1].wait()
        x_cast(my, 1)
        last_copy = None
        for b in range(N_BLOCKS):
            parts = [partial_chunk(my, b * NB + di * NH) for di in range(2)]
            results = []
            for di in range(2):
                prev = rd[di][b]
                prev.wait_recv()
                acc = parts[di] + recv_bufs[di, b].astype(jnp.float32)
                results.append(jnp.maximum(acc * sxw, 0.0))
            if last_copy is not None:
                last_copy.wait()
            staging[:, :NH] = results[0]
            staging[:, NH:] = results[1]
            cp = pltpu.make_async_copy(
                staging,
                out_ref.at[:, pl.ds(b * NB, NB)],
                copy_sem,
            )
            cp.start()
            last_copy = cp
        last_copy.wait()

        for di in range(2):
            for sl in range(N_SLOTS):
                rd_slot[di][sl].wait_send()

    return pl.pallas_call(
        body,
        out_shape=jax.ShapeDtypeStruct((MC, N), jnp.float32),
        in_specs=[
            pl.BlockSpec(memory_space=pltpu.SMEM),
            pl.BlockSpec(memory_space=pl.ANY),
            pl.BlockSpec(memory_space=pl.ANY),
        ],
        out_specs=pl.BlockSpec(memory_space=pl.ANY),
        scratch_shapes=[
            pltpu.VMEM((M, KS), jnp.float8_e5m2),
            pltpu.VMEM((2, MC, KS), jnp.float32),
            pltpu.VMEM((KS, N), jnp.float8_e5m2),
            pltpu.VMEM((2, KS, NB), jnp.float32),
            pltpu.VMEM((2, N_SLOTS, MC, NH), jnp.bfloat16),
            pltpu.VMEM((2, N_BLOCKS, MC, NH), jnp.bfloat16),
            pltpu.VMEM((MC, NB), jnp.float32),
            pltpu.SemaphoreType.DMA((2,)),
            pltpu.SemaphoreType.DMA((2,)),
            pltpu.SemaphoreType.DMA((2, N_SLOTS)),
            pltpu.SemaphoreType.DMA((2, N_BLOCKS)),
            pltpu.SemaphoreType.DMA,
            pltpu.SemaphoreType.REGULAR,
            pltpu.SemaphoreType.REGULAR,
        ],
        compiler_params=pltpu.CompilerParams(
            collective_id=0,
            vmem_limit_bytes=62 * 1024 * 1024,
        ),
    )(sxw, x, w_mat)
